# baseline (device time: 29865 ns/iter reference)
import jax
import jax.numpy as jnp
from jax import lax
from jax.experimental import pallas as pl
from jax.experimental.pallas import tpu as pltpu

N_DEV = 4


def kernel(x, w_mat):
    m, _ = x.shape
    _, n = w_mat.shape
    m_out = m // N_DEV

    def body(x_ref, w_ref, out_ref, comm_ref, send_sems, recv_sems):
        my = lax.axis_index("i")
        left = (my + N_DEV - 1) % N_DEV
        right = (my + 1) % N_DEV

        barrier_sem = pltpu.get_barrier_semaphore()
        for nbr in (left, right):
            pl.semaphore_signal(
                barrier_sem, inc=1,
                device_id=(nbr,), device_id_type=pl.DeviceIdType.MESH,
            )
        pl.semaphore_wait(barrier_sem, 2)

        wb = w_ref[...].astype(jnp.bfloat16)

        def partial_chunk(c):
            xb = x_ref[pl.ds(c * m_out, m_out), :].astype(jnp.bfloat16)
            return lax.dot_general(
                xb, wb, (((1,), (0,)), ((), ())),
                preferred_element_type=jnp.float32,
            )

        c0 = (my + N_DEV - 1) % N_DEV
        comm_ref[0, :, :] = partial_chunk(c0).astype(jnp.bfloat16)

        for s in range(N_DEV - 1):
            send_slot = s % 2
            recv_slot = (s + 1) % 2
            rdma = pltpu.make_async_remote_copy(
                src_ref=comm_ref.at[send_slot],
                dst_ref=comm_ref.at[recv_slot],
                send_sem=send_sems.at[send_slot],
                recv_sem=recv_sems.at[recv_slot],
                device_id=(right,),
                device_id_type=pl.DeviceIdType.MESH,
            )
            rdma.start()
            rdma.wait()

            c = (my + 2 * N_DEV - 2 - s) % N_DEV
            acc = comm_ref[recv_slot, :, :].astype(jnp.float32) + partial_chunk(c)
            if s < N_DEV - 2:
                comm_ref[recv_slot, :, :] = acc.astype(jnp.bfloat16)
            else:
                out_ref[:, :] = jnp.maximum(acc, 0.0)

    return pl.pallas_call(
        body,
        out_shape=jax.ShapeDtypeStruct((m_out, n), jnp.float32),
        in_specs=[
            pl.BlockSpec(memory_space=pltpu.VMEM),
            pl.BlockSpec(memory_space=pltpu.VMEM),
        ],
        out_specs=pl.BlockSpec(memory_space=pltpu.VMEM),
        scratch_shapes=[
            pltpu.VMEM((2, m_out, n), jnp.bfloat16),
            pltpu.SemaphoreType.DMA((2,)),
            pltpu.SemaphoreType.DMA((2,)),
        ],
        compiler_params=pltpu.CompilerParams(collective_id=0),
    )(x, w_mat)


# device time: 17643 ns/iter; 1.6927x vs baseline; 1.6927x over previous
import functools

import jax
import jax.numpy as jnp
from jax import lax
from jax.experimental import pallas as pl
from jax.experimental.pallas import tpu as pltpu

N_DEV = 4

A_RELAY, B_RELAY, A_DIRECT, B_DIRECT, A_SUM, B_SUM = range(6)


def kernel(x, w_mat):
    m, _ = x.shape
    _, n = w_mat.shape
    m_out = m // N_DEV
    n2 = n // 2

    def body(x_ref, w_ref, out_ref, sendbuf, recvbuf, send_sems, recv_sems):
        my = lax.axis_index("i")
        left = (my + N_DEV - 1) % N_DEV
        right = (my + 1) % N_DEV

        barrier_sem = pltpu.get_barrier_semaphore()
        for nbr in (left, right):
            pl.semaphore_signal(
                barrier_sem, inc=1,
                device_id=(nbr,), device_id_type=pl.DeviceIdType.MESH,
            )
        pl.semaphore_wait(barrier_sem, 2)

        wb = w_ref[...].astype(jnp.bfloat16)

        def partial_chunk(c):
            xb = x_ref[pl.ds(c * m_out, m_out), :].astype(jnp.bfloat16)
            return lax.dot_general(
                xb, wb, (((1,), (0,)), ((), ())),
                preferred_element_type=jnp.float32,
            )

        def copy(k, target):
            return pltpu.make_async_remote_copy(
                src_ref=sendbuf.at[k],
                dst_ref=recvbuf.at[k],
                send_sem=send_sems.at[k],
                recv_sem=recv_sems.at[k],
                device_id=(target,),
                device_id_type=pl.DeviceIdType.MESH,
            )

        p_diag = partial_chunk((my + 2) % N_DEV)
        sendbuf[A_RELAY, :, :] = p_diag[:, :n2].astype(jnp.bfloat16)
        r_arelay = copy(A_RELAY, left)
        r_arelay.start()
        sendbuf[B_RELAY, :, :] = p_diag[:, n2:].astype(jnp.bfloat16)
        r_brelay = copy(B_RELAY, right)
        r_brelay.start()

        p_right = partial_chunk(right)
        sendbuf[A_DIRECT, :, :] = p_right[:, :n2].astype(jnp.bfloat16)
        r_adirect = copy(A_DIRECT, right)
        r_adirect.start()

        p_left = partial_chunk(left)
        sendbuf[B_DIRECT, :, :] = p_left[:, n2:].astype(jnp.bfloat16)
        r_bdirect = copy(B_DIRECT, left)
        r_bdirect.start()

        p_own = partial_chunk(my)

        copy(A_RELAY, left).wait_recv()
        a_sum = recvbuf[A_RELAY, :, :].astype(jnp.float32) + p_left[:, :n2]
        sendbuf[A_SUM, :, :] = a_sum.astype(jnp.bfloat16)
        r_asum = copy(A_SUM, left)
        r_asum.start()

        copy(B_RELAY, right).wait_recv()
        b_sum = recvbuf[B_RELAY, :, :].astype(jnp.float32) + p_right[:, n2:]
        sendbuf[B_SUM, :, :] = b_sum.astype(jnp.bfloat16)
        r_bsum = copy(B_SUM, right)
        r_bsum.start()

        copy(A_DIRECT, right).wait_recv()
        copy(A_SUM, left).wait_recv()
        out_ref[:, :n2] = jnp.maximum(
            p_own[:, :n2]
            + recvbuf[A_DIRECT, :, :].astype(jnp.float32)
            + recvbuf[A_SUM, :, :].astype(jnp.float32),
            0.0,
        )
        copy(B_DIRECT, left).wait_recv()
        copy(B_SUM, right).wait_recv()
        out_ref[:, n2:] = jnp.maximum(
            p_own[:, n2:]
            + recvbuf[B_DIRECT, :, :].astype(jnp.float32)
            + recvbuf[B_SUM, :, :].astype(jnp.float32),
            0.0,
        )

        for r in (r_arelay, r_brelay, r_adirect, r_bdirect, r_asum, r_bsum):
            r.wait_send()

        @functools.partial(pl.run_scoped, exit_sem=pltpu.SemaphoreType.REGULAR)
        def _(exit_sem):
            for nbr in (left, right):
                pl.semaphore_signal(
                    exit_sem, inc=1,
                    device_id=(nbr,), device_id_type=pl.DeviceIdType.MESH,
                )
            pl.semaphore_wait(exit_sem, 2)

    return pl.pallas_call(
        body,
        out_shape=jax.ShapeDtypeStruct((m_out, n), jnp.float32),
        in_specs=[
            pl.BlockSpec(memory_space=pltpu.VMEM),
            pl.BlockSpec(memory_space=pltpu.VMEM),
        ],
        out_specs=pl.BlockSpec(memory_space=pltpu.VMEM),
        scratch_shapes=[
            pltpu.VMEM((6, m_out, n2), jnp.bfloat16),
            pltpu.VMEM((6, m_out, n2), jnp.bfloat16),
            pltpu.SemaphoreType.DMA((6,)),
            pltpu.SemaphoreType.DMA((6,)),
        ],
        compiler_params=pltpu.CompilerParams(collective_id=0),
    )(x, w_mat)
